# Initial kernel scaffold; baseline (speedup 1.0000x reference)
import functools

import jax
import jax.numpy as jnp
from jax import lax
from jax.experimental import pallas as pl
from jax.experimental.pallas import tpu as pltpu
from jax.experimental.pallas import tpu_sc as plsc

_NC = 2
_NS = 16
_NW = _NC * _NS

_CH = 100


def _xw_kernel(x_ref, w_ref, o_ref):
    o_ref[...] = jnp.dot(x_ref[...], w_ref[...],
                         preferred_element_type=jnp.float32)


def _mlp_kernel(g_ref, da_ref, w1da_ref, s1_ref, c1_ref, w2_ref, s2_ref,
                c2_ref, o_ref):
    d = g_ref.shape[1]
    t = g_ref[...] + jnp.dot(da_ref[...], w1da_ref[...],
                             preferred_element_type=jnp.float32)
    u = t * s1_ref[...] + c1_ref[...]
    u = u * jax.nn.sigmoid(u)
    v = jnp.dot(u, w2_ref[...], preferred_element_type=jnp.float32)
    v = v * s2_ref[...] + c2_ref[...]
    o_ref[...] = jax.nn.sigmoid(v[:, :d]) * jax.nn.softplus(v[:, d:])


def _psum_kernel(p_ref, o_ref):
    o_ref[...] = p_ref[0] + p_ref[1]


def kernel(x, edge_index, encoded_dists, encoded_angles, W1, b1, g1, bt1,
           m1, v1, W2, b2, g2, bt2, m2, v2):
    n, din = x.shape
    e = edge_index.shape[1]
    dout = W1.shape[1]

    chunks = e // _CH
    assert chunks * _CH == e and chunks % _NW == 0
    cpw = chunks // _NW
    cpc = chunks // _NC
    cpw_sc = cpc // _NS
    rps = n // _NS
    assert rps * _NS == n

    s1 = g1 * lax.rsqrt(v1 + 1e-5)
    c1 = (b1 - m1) * s1 + bt1
    s2 = g2 * lax.rsqrt(v2 + 1e-5)
    c2 = (b2 - m2) * s2 + bt2

    da = jnp.concatenate([encoded_dists, encoded_angles], axis=1)
    dda = da.shape[1]
    idx_in = edge_index[0].reshape(chunks, _CH)
    idx_out = edge_index[1].reshape(chunks, _CH)

    nb = 10
    xw = pl.pallas_call(
        _xw_kernel,
        grid=(nb,),
        in_specs=[pl.BlockSpec((n // nb, din), lambda i: (i, 0)),
                  pl.BlockSpec((din, dout), lambda i: (0, 0))],
        out_specs=pl.BlockSpec((n // nb, dout), lambda i: (i, 0)),
        out_shape=jax.ShapeDtypeStruct((n, dout), jnp.float32),
    )(x, W1[:din])

    mesh = plsc.VectorSubcoreMesh(core_axis_name="c", subcore_axis_name="s")

    @functools.partial(
        pl.kernel, mesh=mesh,
        out_type=jax.ShapeDtypeStruct((e, dout), jnp.float32),
        scratch_types=[pltpu.VMEM((_CH,), jnp.int32),
                       pltpu.VMEM((_CH, dout), jnp.float32),
                       pltpu.SemaphoreType.DMA],
    )
    def _gather_sc(table_hbm, idx_hbm, out_hbm, idx_v, rows_v, sem):
        wid = lax.axis_index("s") * _NC + lax.axis_index("c")

        def body(j, carry):
            chunk = wid * cpw + j
            pltpu.sync_copy(idx_hbm.at[chunk], idx_v)
            pltpu.async_copy(table_hbm.at[idx_v], rows_v, sem).wait()
            pltpu.sync_copy(rows_v, out_hbm.at[pl.ds(chunk * _CH, _CH)])
            return carry

        lax.fori_loop(0, cpw, body, 0)

    g = _gather_sc(xw, idx_in)

    be = 2000
    eb = e // be
    msg = pl.pallas_call(
        _mlp_kernel,
        grid=(eb,),
        in_specs=[pl.BlockSpec((be, dout), lambda i: (i, 0)),
                  pl.BlockSpec((be, dda), lambda i: (i, 0)),
                  pl.BlockSpec((dda, dout), lambda i: (0, 0)),
                  pl.BlockSpec((1, dout), lambda i: (0, 0)),
                  pl.BlockSpec((1, dout), lambda i: (0, 0)),
                  pl.BlockSpec((dout, 2 * dout), lambda i: (0, 0)),
                  pl.BlockSpec((1, 2 * dout), lambda i: (0, 0)),
                  pl.BlockSpec((1, 2 * dout), lambda i: (0, 0))],
        out_specs=pl.BlockSpec((be, dout), lambda i: (i, 0)),
        out_shape=jax.ShapeDtypeStruct((e, dout), jnp.float32),
    )(g, da, W1[din:], s1[None], c1[None], W2, s2[None], c2[None])

    @functools.partial(
        pl.kernel, mesh=mesh,
        out_type=jax.ShapeDtypeStruct((_NC, n, dout), jnp.float32),
        scratch_types=[pltpu.VMEM((_CH,), jnp.int32),
                       pltpu.VMEM((_CH, dout), jnp.float32),
                       pltpu.VMEM_SHARED((n, dout), jnp.float32),
                       pltpu.SemaphoreType.DMA],
    )
    def _scatter_sc(msg_hbm, idx_hbm, zero_hbm, part_hbm, idx_v, msg_v, acc,
                    sem):
        c = lax.axis_index("c")
        s = lax.axis_index("s")
        pltpu.sync_copy(zero_hbm.at[pl.ds(s * rps, rps)],
                        acc.at[pl.ds(s * rps, rps)])
        plsc.subcore_barrier()

        def body(j, carry):
            chunk = c * cpc + s * cpw_sc + j
            pltpu.sync_copy(idx_hbm.at[chunk], idx_v)
            pltpu.sync_copy(msg_hbm.at[pl.ds(chunk * _CH, _CH)], msg_v)
            pltpu.sync_copy(msg_v, acc.at[idx_v], add=True)
            return carry

        lax.fori_loop(0, cpw_sc, body, 0)
        plsc.subcore_barrier()
        pltpu.sync_copy(acc.at[pl.ds(s * rps, rps)],
                        part_hbm.at[c, pl.ds(s * rps, rps)])

    parts = _scatter_sc(msg, idx_out, jnp.zeros((n, dout), jnp.float32))

    out = pl.pallas_call(
        _psum_kernel,
        grid=(nb,),
        in_specs=[pl.BlockSpec((_NC, n // nb, dout), lambda i: (0, i, 0))],
        out_specs=pl.BlockSpec((n // nb, dout), lambda i: (i, 0)),
        out_shape=jax.ShapeDtypeStruct((n, dout), jnp.float32),
    )(parts)
    return out


# speedup vs baseline: 2.8524x; 2.8524x over previous
import functools

import jax
import jax.numpy as jnp
from jax import lax
from jax.experimental import pallas as pl
from jax.experimental.pallas import tpu as pltpu
from jax.experimental.pallas import tpu_sc as plsc

_NC = 2
_NS = 16
_NW = _NC * _NS

_CH = 100


def _xw_kernel(x_ref, w_ref, o_ref):
    o_ref[...] = jnp.dot(x_ref[...], w_ref[...],
                         preferred_element_type=jnp.float32)


def _mlp_kernel(g_ref, da_ref, w1da_ref, s1_ref, c1_ref, w2_ref, s2_ref,
                c2_ref, o_ref):
    d = g_ref.shape[1]
    t = g_ref[...] + jnp.dot(da_ref[...], w1da_ref[...],
                             preferred_element_type=jnp.float32)
    u = t * s1_ref[...] + c1_ref[...]
    u = u * jax.nn.sigmoid(u)
    v = jnp.dot(u, w2_ref[...], preferred_element_type=jnp.float32)
    v = v * s2_ref[...] + c2_ref[...]
    o_ref[...] = jax.nn.sigmoid(v[:, :d]) * jax.nn.softplus(v[:, d:])


def _psum_kernel(p_ref, o_ref):
    o_ref[...] = p_ref[0] + p_ref[1]


def kernel(x, edge_index, encoded_dists, encoded_angles, W1, b1, g1, bt1,
           m1, v1, W2, b2, g2, bt2, m2, v2):
    n, din = x.shape
    e = edge_index.shape[1]
    dout = W1.shape[1]

    chunks = e // _CH
    assert chunks * _CH == e and chunks % _NW == 0
    cpw = chunks // _NW
    cpc = chunks // _NC
    cpw_sc = cpc // _NS
    rps = n // _NS
    assert rps * _NS == n

    s1 = g1 * lax.rsqrt(v1 + 1e-5)
    c1 = (b1 - m1) * s1 + bt1
    s2 = g2 * lax.rsqrt(v2 + 1e-5)
    c2 = (b2 - m2) * s2 + bt2

    da = jnp.concatenate([encoded_dists, encoded_angles], axis=1)
    dda = da.shape[1]
    idx_in = edge_index[0].reshape(chunks, _CH)
    idx_out = edge_index[1].reshape(chunks, _CH)

    nb = 10
    xw = pl.pallas_call(
        _xw_kernel,
        grid=(nb,),
        in_specs=[pl.BlockSpec((n // nb, din), lambda i: (i, 0)),
                  pl.BlockSpec((din, dout), lambda i: (0, 0))],
        out_specs=pl.BlockSpec((n // nb, dout), lambda i: (i, 0)),
        out_shape=jax.ShapeDtypeStruct((n, dout), jnp.float32),
    )(x, W1[:din])

    mesh = plsc.VectorSubcoreMesh(core_axis_name="c", subcore_axis_name="s")

    @functools.partial(
        pl.kernel, mesh=mesh,
        out_type=jax.ShapeDtypeStruct((e, dout), jnp.float32),
        scratch_types=[pltpu.VMEM((_CH,), jnp.int32),
                       pltpu.VMEM((_CH, dout), jnp.float32),
                       pltpu.SemaphoreType.DMA],
        compiler_params=pltpu.CompilerParams(use_tc_tiling_on_sc=False),
    )
    def _gather_sc(table_hbm, idx_hbm, out_hbm, idx_v, rows_v, sem):
        wid = lax.axis_index("s") * _NC + lax.axis_index("c")

        def body(j, carry):
            chunk = wid * cpw + j
            pltpu.sync_copy(idx_hbm.at[chunk], idx_v)
            pltpu.async_copy(table_hbm.at[idx_v], rows_v, sem).wait()
            pltpu.sync_copy(rows_v, out_hbm.at[pl.ds(chunk * _CH, _CH)])
            return carry

        lax.fori_loop(0, cpw, body, 0)

    g = _gather_sc(xw, idx_in)

    be = 2000
    eb = e // be
    msg = pl.pallas_call(
        _mlp_kernel,
        grid=(eb,),
        in_specs=[pl.BlockSpec((be, dout), lambda i: (i, 0)),
                  pl.BlockSpec((be, dda), lambda i: (i, 0)),
                  pl.BlockSpec((dda, dout), lambda i: (0, 0)),
                  pl.BlockSpec((1, dout), lambda i: (0, 0)),
                  pl.BlockSpec((1, dout), lambda i: (0, 0)),
                  pl.BlockSpec((dout, 2 * dout), lambda i: (0, 0)),
                  pl.BlockSpec((1, 2 * dout), lambda i: (0, 0)),
                  pl.BlockSpec((1, 2 * dout), lambda i: (0, 0))],
        out_specs=pl.BlockSpec((be, dout), lambda i: (i, 0)),
        out_shape=jax.ShapeDtypeStruct((e, dout), jnp.float32),
    )(g, da, W1[din:], s1[None], c1[None], W2, s2[None], c2[None])

    @functools.partial(
        pl.kernel, mesh=mesh,
        out_type=jax.ShapeDtypeStruct((_NC, n, dout), jnp.float32),
        scratch_types=[pltpu.VMEM((_CH,), jnp.int32),
                       pltpu.VMEM((_CH, dout), jnp.float32),
                       pltpu.VMEM_SHARED((n, dout), jnp.float32),
                       pltpu.SemaphoreType.DMA],
        compiler_params=pltpu.CompilerParams(use_tc_tiling_on_sc=False),
    )
    def _scatter_sc(msg_hbm, idx_hbm, zero_hbm, part_hbm, idx_v, msg_v, acc,
                    sem):
        c = lax.axis_index("c")
        s = lax.axis_index("s")
        pltpu.sync_copy(zero_hbm.at[pl.ds(s * rps, rps)],
                        acc.at[pl.ds(s * rps, rps)])
        plsc.subcore_barrier()

        def body(j, carry):
            chunk = c * cpc + s * cpw_sc + j
            pltpu.sync_copy(idx_hbm.at[chunk], idx_v)
            pltpu.sync_copy(msg_hbm.at[pl.ds(chunk * _CH, _CH)], msg_v)
            pltpu.sync_copy(msg_v, acc.at[idx_v], add=True)
            return carry

        lax.fori_loop(0, cpw_sc, body, 0)
        plsc.subcore_barrier()
        pltpu.sync_copy(acc.at[pl.ds(s * rps, rps)],
                        part_hbm.at[c, pl.ds(s * rps, rps)])

    parts = _scatter_sc(msg, idx_out, jnp.zeros((n, dout), jnp.float32))

    out = pl.pallas_call(
        _psum_kernel,
        grid=(nb,),
        in_specs=[pl.BlockSpec((_NC, n // nb, dout), lambda i: (0, i, 0))],
        out_specs=pl.BlockSpec((n // nb, dout), lambda i: (i, 0)),
        out_shape=jax.ShapeDtypeStruct((n, dout), jnp.float32),
    )(parts)
    return out


# speedup vs baseline: 3.2182x; 1.1282x over previous
import functools

import jax
import jax.numpy as jnp
from jax import lax
from jax.experimental import pallas as pl
from jax.experimental.pallas import tpu as pltpu
from jax.experimental.pallas import tpu_sc as plsc

_NC = 2
_NS = 16
_NW = _NC * _NS

_CH = 100
_S = 4


def _xw_kernel(x_ref, w_ref, o_ref):
    o_ref[...] = jnp.dot(x_ref[...], w_ref[...],
                         preferred_element_type=jnp.float32)


def _mlp_kernel(g_ref, d_ref, a_ref, w1d_ref, w1a_ref, s1_ref, c1_ref,
                w2_ref, s2_ref, c2_ref, o_ref):
    d = g_ref.shape[1]
    bf = jnp.bfloat16
    t = (g_ref[...]
         + jnp.dot(d_ref[...].astype(bf), w1d_ref[...],
                   preferred_element_type=jnp.float32)
         + jnp.dot(a_ref[...].astype(bf), w1a_ref[...],
                   preferred_element_type=jnp.float32))
    u = t * s1_ref[...] + c1_ref[...]
    u = u * jax.nn.sigmoid(u)
    v = jnp.dot(u.astype(bf), w2_ref[...], preferred_element_type=jnp.float32)
    v = v * s2_ref[...] + c2_ref[...]
    o_ref[...] = jax.nn.sigmoid(v[:, :d]) * jax.nn.softplus(v[:, d:])


def _psum_kernel(p1_ref, p2_ref, o_ref):
    o_ref[...] = ((p1_ref[0] + p1_ref[1]) + (p2_ref[0] + p2_ref[1]))


def kernel(x, edge_index, encoded_dists, encoded_angles, W1, b1, g1, bt1,
           m1, v1, W2, b2, g2, bt2, m2, v2):
    n, din = x.shape
    e = edge_index.shape[1]
    dout = W1.shape[1]
    dg = encoded_dists.shape[1]
    bf = jnp.bfloat16

    es = e // _S
    chunks_s = es // _CH
    cpw_g = chunks_s // _NW
    nbg = 5
    ngr_g = cpw_g // nbg
    cpw_s = chunks_s // _NS
    nbs = 2
    ngr_s = cpw_s // nbs
    rps = n // _NS
    assert es * _S == e and chunks_s * _CH == es
    assert ngr_g * nbg == cpw_g and ngr_s * nbs == cpw_s and rps * _NS == n

    s1 = g1 * lax.rsqrt(v1 + 1e-5)
    c1 = (b1 - m1) * s1 + bt1
    s2 = g2 * lax.rsqrt(v2 + 1e-5)
    c2 = (b2 - m2) * s2 + bt2

    src = edge_index[0].reshape(_S, _NW, cpw_g, _CH)
    dst = edge_index[1].reshape(_S, _NS, cpw_s, _CH)

    nrb = 10
    xw = pl.pallas_call(
        _xw_kernel,
        grid=(nrb,),
        in_specs=[pl.BlockSpec((n // nrb, din), lambda i: (i, 0)),
                  pl.BlockSpec((din, dout), lambda i: (0, 0))],
        out_specs=pl.BlockSpec((n // nrb, dout), lambda i: (i, 0)),
        out_shape=jax.ShapeDtypeStruct((n, dout), jnp.float32),
    )(x, W1[:din])

    mesh = plsc.VectorSubcoreMesh(core_axis_name="c", subcore_axis_name="s")
    sc_params = pltpu.CompilerParams(use_tc_tiling_on_sc=False)

    def make_gather(si):
        @functools.partial(
            pl.kernel, mesh=mesh,
            out_type=jax.ShapeDtypeStruct((es, dout), jnp.float32),
            scratch_types=[pltpu.VMEM((cpw_g, _CH), jnp.int32),
                           pltpu.VMEM((nbg, _CH, dout), jnp.float32),
                           pltpu.SemaphoreType.DMA((nbg,)),
                           pltpu.SemaphoreType.DMA((nbg,))],
            compiler_params=sc_params,
        )
        def _gather_sc(table_hbm, idx_hbm, out_hbm, idx_all, rows, sem_g,
                       sem_s):
            wid = lax.axis_index("s") * _NC + lax.axis_index("c")
            base = wid * cpw_g
            pltpu.sync_copy(idx_hbm.at[si, wid], idx_all)
            for b in range(nbg):
                pltpu.async_copy(table_hbm.at[idx_all.at[b]], rows.at[b],
                                 sem_g.at[b])

            def group(gi, carry):
                for b in range(nbg):
                    j = gi * nbg + b
                    pltpu.make_async_copy(table_hbm.at[idx_all.at[j]],
                                          rows.at[b], sem_g.at[b]).wait()
                    pltpu.async_copy(
                        rows.at[b], out_hbm.at[pl.ds((base + j) * _CH, _CH)],
                        sem_s.at[b])
                for b in range(nbg):
                    j = gi * nbg + b
                    jn = j + nbg

                    @pl.when(jn < cpw_g)
                    def _():
                        pltpu.make_async_copy(
                            rows.at[b],
                            out_hbm.at[pl.ds((base + j) * _CH, _CH)],
                            sem_s.at[b]).wait()
                        pltpu.async_copy(table_hbm.at[idx_all.at[jn]],
                                         rows.at[b], sem_g.at[b])
                return carry

            lax.fori_loop(0, ngr_g, group, 0)
            for b in range(nbg):
                j = (ngr_g - 1) * nbg + b
                pltpu.make_async_copy(
                    rows.at[b], out_hbm.at[pl.ds((base + j) * _CH, _CH)],
                    sem_s.at[b]).wait()

        return _gather_sc

    be = 2000
    eb = es // be

    def mlp_call(g_seg, si):
        off = si * eb

        def seg_map(i, o=off):
            return (o + i, 0)

        zmap = lambda i: (0, 0)
        return pl.pallas_call(
            _mlp_kernel,
            grid=(eb,),
            in_specs=[pl.BlockSpec((be, dout), lambda i: (i, 0)),
                      pl.BlockSpec((be, dg), seg_map),
                      pl.BlockSpec((be, dg), seg_map),
                      pl.BlockSpec((dg, dout), zmap),
                      pl.BlockSpec((dg, dout), zmap),
                      pl.BlockSpec((1, dout), zmap),
                      pl.BlockSpec((1, dout), zmap),
                      pl.BlockSpec((dout, 2 * dout), zmap),
                      pl.BlockSpec((1, 2 * dout), zmap),
                      pl.BlockSpec((1, 2 * dout), zmap)],
            out_specs=pl.BlockSpec((be, dout), lambda i: (i, 0)),
            out_shape=jax.ShapeDtypeStruct((es, dout), jnp.float32),
        )(g_seg, encoded_dists, encoded_angles,
          W1[din:din + dg].astype(bf), W1[din + dg:].astype(bf),
          s1[None], c1[None], W2.astype(bf), s2[None], c2[None])

    def make_scatter(k):
        @functools.partial(
            pl.kernel, mesh=mesh,
            out_type=jax.ShapeDtypeStruct((_NC, n, dout), jnp.float32),
            scratch_types=[pltpu.VMEM((cpw_s, _CH), jnp.int32),
                           pltpu.VMEM((nbs, _CH, dout), jnp.float32),
                           pltpu.VMEM_SHARED((n, dout), jnp.float32),
                           pltpu.SemaphoreType.DMA((nbs,)),
                           pltpu.SemaphoreType.DMA((nbs,))],
            compiler_params=sc_params,
        )
        def _scatter_sc(msg_a, msg_b, idx_hbm, zero_hbm, part_hbm, idx_all,
                        msg_v, acc, sem_l, sem_a):
            c = lax.axis_index("c")
            s = lax.axis_index("s")
            pltpu.sync_copy(zero_hbm.at[pl.ds(s * rps, rps)],
                            acc.at[pl.ds(s * rps, rps)])

            def run(msg_hbm, si):
                base = s * cpw_s
                pltpu.sync_copy(idx_hbm.at[si, s], idx_all)
                plsc.subcore_barrier()
                for b in range(nbs):
                    pltpu.async_copy(
                        msg_hbm.at[pl.ds((base + b) * _CH, _CH)],
                        msg_v.at[b], sem_l.at[b])

                def group(gi, carry):
                    for b in range(nbs):
                        j = gi * nbs + b
                        pltpu.make_async_copy(
                            msg_hbm.at[pl.ds((base + j) * _CH, _CH)],
                            msg_v.at[b], sem_l.at[b]).wait()
                        pltpu.async_copy(msg_v.at[b], acc.at[idx_all.at[j]],
                                         sem_a.at[b], add=True)
                    for b in range(nbs):
                        j = gi * nbs + b
                        jn = j + nbs

                        @pl.when(jn < cpw_s)
                        def _():
                            pltpu.make_async_copy(msg_v.at[b],
                                                  acc.at[idx_all.at[j]],
                                                  sem_a.at[b]).wait()
                            pltpu.async_copy(
                                msg_hbm.at[pl.ds((base + jn) * _CH, _CH)],
                                msg_v.at[b], sem_l.at[b])
                    return carry

                lax.fori_loop(0, ngr_s, group, 0)
                for b in range(nbs):
                    j = (ngr_s - 1) * nbs + b
                    pltpu.make_async_copy(msg_v.at[b], acc.at[idx_all.at[j]],
                                          sem_a.at[b]).wait()
                plsc.subcore_barrier()

            @pl.when(c == 0)
            def _():
                run(msg_a, 2 * k)

            @pl.when(c == 1)
            def _():
                run(msg_b, 2 * k + 1)

            pltpu.sync_copy(acc.at[pl.ds(s * rps, rps)],
                            part_hbm.at[c, pl.ds(s * rps, rps)])

        return _scatter_sc

    zeros = jnp.zeros((n, dout), jnp.float32)

    msgs = []
    for si in range(_S):
        g_seg = make_gather(si)(xw, src)
        msgs.append(mlp_call(g_seg, si))

    p1 = make_scatter(0)(msgs[0], msgs[1], dst, zeros)
    p2 = make_scatter(1)(msgs[2], msgs[3], dst, zeros)

    out = pl.pallas_call(
        _psum_kernel,
        grid=(nrb,),
        in_specs=[pl.BlockSpec((_NC, n // nrb, dout), lambda i: (0, i, 0)),
                  pl.BlockSpec((_NC, n // nrb, dout), lambda i: (0, i, 0))],
        out_specs=pl.BlockSpec((n // nrb, dout), lambda i: (i, 0)),
        out_shape=jax.ShapeDtypeStruct((n, dout), jnp.float32),
    )(p1, p2)
    return out


# speedup vs baseline: 4.1276x; 1.2826x over previous
import functools

import jax
import jax.numpy as jnp
from jax import lax
from jax.experimental import pallas as pl
from jax.experimental.pallas import tpu as pltpu
from jax.experimental.pallas import tpu_sc as plsc

_NC = 2
_NS = 16
_NW = _NC * _NS

_CHG = 100
_CHS = 125
_S = 4


def _xw_kernel(x_ref, w_ref, o_ref):
    o_ref[...] = jnp.dot(x_ref[...], w_ref[...],
                         preferred_element_type=jnp.float32)


def _mlp_kernel(g_ref, da_ref, w1da_ref, s1_ref, c1_ref, w2_ref, s2_ref,
                c2_ref, o_ref):
    d = g_ref.shape[1]
    bf = jnp.bfloat16
    t = g_ref[...] + jnp.dot(da_ref[...].astype(bf), w1da_ref[...],
                             preferred_element_type=jnp.float32)
    u = t * s1_ref[...] + c1_ref[...]
    u = u * jax.nn.sigmoid(u)
    v = jnp.dot(u.astype(bf), w2_ref[...], preferred_element_type=jnp.float32)
    v = v * s2_ref[...] + c2_ref[...]
    o_ref[...] = jax.nn.sigmoid(v[:, :d]) * jax.nn.softplus(v[:, d:])


def _psum_kernel(p0_ref, p1_ref, p2_ref, p3_ref, o_ref):
    o_ref[...] = ((p0_ref[0] + p0_ref[1]) + (p1_ref[0] + p1_ref[1])
                  + (p2_ref[0] + p2_ref[1]) + (p3_ref[0] + p3_ref[1]))


def kernel(x, edge_index, encoded_dists, encoded_angles, W1, b1, g1, bt1,
           m1, v1, W2, b2, g2, bt2, m2, v2):
    n, din = x.shape
    e = edge_index.shape[1]
    dout = W1.shape[1]
    dg = encoded_dists.shape[1]
    bf = jnp.bfloat16

    es = e // _S
    gch = es // _CHG
    cpw_g = gch // _NW
    nbg = 5
    ngr_g = cpw_g // nbg
    sch = es // _CHS
    cpw_s = sch // _NW
    nbs = 2
    ngr_s = cpw_s // nbs
    rps = n // _NS
    assert es * _S == e and gch * _CHG == es and sch * _CHS == es
    assert ngr_g * nbg == cpw_g and ngr_s * nbs == cpw_s and rps * _NS == n

    s1 = g1 * lax.rsqrt(v1 + 1e-5)
    c1 = (b1 - m1) * s1 + bt1
    s2 = g2 * lax.rsqrt(v2 + 1e-5)
    c2 = (b2 - m2) * s2 + bt2

    da = jnp.concatenate([encoded_dists, encoded_angles], axis=1)
    src = edge_index[0].reshape(_S, _NW, cpw_g, _CHG)
    dst = edge_index[1].reshape(_S, _NW, cpw_s, _CHS)

    nrb = 10
    xw = pl.pallas_call(
        _xw_kernel,
        grid=(nrb,),
        in_specs=[pl.BlockSpec((n // nrb, din), lambda i: (i, 0)),
                  pl.BlockSpec((din, dout), lambda i: (0, 0))],
        out_specs=pl.BlockSpec((n // nrb, dout), lambda i: (i, 0)),
        out_shape=jax.ShapeDtypeStruct((n, dout), jnp.float32),
    )(x, W1[:din])

    mesh = plsc.VectorSubcoreMesh(core_axis_name="c", subcore_axis_name="s")
    sc_params = pltpu.CompilerParams(use_tc_tiling_on_sc=False)

    def make_gather(si):
        @functools.partial(
            pl.kernel, mesh=mesh,
            out_type=jax.ShapeDtypeStruct((es, dout), jnp.float32),
            scratch_types=[pltpu.VMEM((cpw_g, _CHG), jnp.int32),
                           pltpu.VMEM((nbg, _CHG, dout), jnp.float32),
                           pltpu.SemaphoreType.DMA((nbg,)),
                           pltpu.SemaphoreType.DMA((nbg,))],
            compiler_params=sc_params,
        )
        def _gather_sc(table_hbm, idx_hbm, out_hbm, idx_all, rows, sem_g,
                       sem_s):
            wid = lax.axis_index("s") * _NC + lax.axis_index("c")
            base = wid * cpw_g
            pltpu.sync_copy(idx_hbm.at[si, wid], idx_all)
            for b in range(nbg):
                pltpu.async_copy(table_hbm.at[idx_all.at[b]], rows.at[b],
                                 sem_g.at[b])

            def group(gi, carry):
                for b in range(nbg):
                    j = gi * nbg + b
                    pltpu.make_async_copy(table_hbm.at[idx_all.at[j]],
                                          rows.at[b], sem_g.at[b]).wait()
                    pltpu.async_copy(
                        rows.at[b],
                        out_hbm.at[pl.ds((base + j) * _CHG, _CHG)],
                        sem_s.at[b])
                for b in range(nbg):
                    j = gi * nbg + b
                    jn = j + nbg

                    @pl.when(jn < cpw_g)
                    def _():
                        pltpu.make_async_copy(
                            rows.at[b],
                            out_hbm.at[pl.ds((base + j) * _CHG, _CHG)],
                            sem_s.at[b]).wait()
                        pltpu.async_copy(table_hbm.at[idx_all.at[jn]],
                                         rows.at[b], sem_g.at[b])
                return carry

            lax.fori_loop(0, ngr_g, group, 0)
            for b in range(nbg):
                j = (ngr_g - 1) * nbg + b
                pltpu.make_async_copy(
                    rows.at[b], out_hbm.at[pl.ds((base + j) * _CHG, _CHG)],
                    sem_s.at[b]).wait()

        return _gather_sc

    be = 4000
    eb = es // be
    assert eb * be == es

    def mlp_call(g_seg, si):
        off = si * eb

        def seg_map(i, o=off):
            return (o + i, 0)

        zmap = lambda i: (0, 0)
        return pl.pallas_call(
            _mlp_kernel,
            grid=(eb,),
            in_specs=[pl.BlockSpec((be, dout), lambda i: (i, 0)),
                      pl.BlockSpec((be, 2 * dg), seg_map),
                      pl.BlockSpec((2 * dg, dout), zmap),
                      pl.BlockSpec((1, dout), zmap),
                      pl.BlockSpec((1, dout), zmap),
                      pl.BlockSpec((dout, 2 * dout), zmap),
                      pl.BlockSpec((1, 2 * dout), zmap),
                      pl.BlockSpec((1, 2 * dout), zmap)],
            out_specs=pl.BlockSpec((be, dout), lambda i: (i, 0)),
            out_shape=jax.ShapeDtypeStruct((es, dout), jnp.float32),
        )(g_seg, da, W1[din:].astype(bf),
          s1[None], c1[None], W2.astype(bf), s2[None], c2[None])

    def make_scatter(si):
        @functools.partial(
            pl.kernel, mesh=mesh,
            out_type=jax.ShapeDtypeStruct((_NC, n, dout), jnp.float32),
            scratch_types=[pltpu.VMEM((cpw_s, _CHS), jnp.int32),
                           pltpu.VMEM((nbs, _CHS, dout), jnp.float32),
                           pltpu.VMEM_SHARED((n, dout), jnp.float32),
                           pltpu.SemaphoreType.DMA((nbs,)),
                           pltpu.SemaphoreType.DMA((nbs,))],
            compiler_params=sc_params,
        )
        def _scatter_sc(msg_hbm, idx_hbm, zero_hbm, part_hbm, idx_all,
                        msg_v, acc, sem_l, sem_a):
            c = lax.axis_index("c")
            s = lax.axis_index("s")
            wid = s * _NC + c
            base = wid * cpw_s
            pltpu.sync_copy(idx_hbm.at[si, wid], idx_all)
            pltpu.sync_copy(zero_hbm.at[pl.ds(s * rps, rps)],
                            acc.at[pl.ds(s * rps, rps)])
            plsc.subcore_barrier()
            for b in range(nbs):
                pltpu.async_copy(msg_hbm.at[pl.ds((base + b) * _CHS, _CHS)],
                                 msg_v.at[b], sem_l.at[b])

            def group(gi, carry):
                for b in range(nbs):
                    j = gi * nbs + b
                    pltpu.make_async_copy(
                        msg_hbm.at[pl.ds((base + j) * _CHS, _CHS)],
                        msg_v.at[b], sem_l.at[b]).wait()
                    pltpu.async_copy(msg_v.at[b], acc.at[idx_all.at[j]],
                                     sem_a.at[b], add=True)
                for b in range(nbs):
                    j = gi * nbs + b
                    jn = j + nbs

                    @pl.when(jn < cpw_s)
                    def _():
                        pltpu.make_async_copy(msg_v.at[b],
                                              acc.at[idx_all.at[j]],
                                              sem_a.at[b]).wait()
                        pltpu.async_copy(
                            msg_hbm.at[pl.ds((base + jn) * _CHS, _CHS)],
                            msg_v.at[b], sem_l.at[b])
                return carry

            lax.fori_loop(0, ngr_s, group, 0)
            for b in range(nbs):
                j = (ngr_s - 1) * nbs + b
                pltpu.make_async_copy(msg_v.at[b], acc.at[idx_all.at[j]],
                                      sem_a.at[b]).wait()
            plsc.subcore_barrier()
            pltpu.sync_copy(acc.at[pl.ds(s * rps, rps)],
                            part_hbm.at[c, pl.ds(s * rps, rps)])

        return _scatter_sc

    zeros = jnp.zeros((n, dout), jnp.float32)

    parts = []
    for si in range(_S):
        g_seg = make_gather(si)(xw, src)
        msg_seg = mlp_call(g_seg, si)
        parts.append(make_scatter(si)(msg_seg, dst, zeros))

    out = pl.pallas_call(
        _psum_kernel,
        grid=(nrb,),
        in_specs=[pl.BlockSpec((_NC, n // nrb, dout), lambda i: (0, i, 0))
                  for _ in range(_S)],
        out_specs=pl.BlockSpec((n // nrb, dout), lambda i: (i, 0)),
        out_shape=jax.ShapeDtypeStruct((n, dout), jnp.float32),
    )(*parts)
    return out
